# baseline (speedup 1.0000x reference)
import functools

import jax
import jax.numpy as jnp
from jax import lax
from jax.experimental import pallas as pl
from jax.experimental.pallas import tpu as pltpu
from jax.experimental.pallas import tpu_sc as plsc

NUM_USERS = 100000
NUM_ITEMS = 100000
EMB = 64
N_FEAT = 4
FEAT_VOCAB = 1000
FEAT_DIM = 32
LATENT = 64
B = 16384

NC, NS, L = 2, 16, 16
NW = NC * NS
BPW = B // NW
CHUNK = 64
NCHUNK = BPW // CHUNK
GROUPS = CHUNK // L
WU = 256
WF = 128
KU = 9


def _sc_body(uid_h, iid_h, ufeat_h, up_h, ip_h, fup_h, fip_h, mean_h, out_h,
             uid_v, iid_v, uf_v, fuidx_v,
             UP_v, IP_v, FU_v, FI_v,
             mean_v, sbuf_v, out_v, sem):
    wid = lax.axis_index("s") * NC + lax.axis_index("c")
    base = wid * BPW
    lanes = lax.iota(jnp.int32, L)

    pltpu.sync_copy(mean_h, mean_v)

    def chunk_body(ci, _):
        cb = base + ci * CHUNK
        pltpu.sync_copy(uid_h.at[pl.ds(cb, CHUNK)], uid_v)
        pltpu.sync_copy(iid_h.at[pl.ds(cb, CHUNK)], iid_v)
        pltpu.sync_copy(ufeat_h.at[pl.ds(cb * N_FEAT, CHUNK * N_FEAT)], uf_v)

        def idx_body(j, _):
            for f in range(N_FEAT):
                uv = plsc.load_gather(uf_v, [(j * L + lanes) * N_FEAT + f])
                fuidx_v[pl.ds(f * CHUNK + j * L, L)] = uv
            return 0

        lax.fori_loop(0, CHUNK // L, idx_body, 0)

        cps = [
            pltpu.async_copy(up_h.at[uid_v], UP_v, sem),
            pltpu.async_copy(ip_h.at[iid_v], IP_v, sem),
            pltpu.async_copy(fup_h.at[fuidx_v], FU_v, sem),
            pltpu.async_copy(fip_h.at[iid_v], FI_v, sem),
        ]
        for cp in cps:
            cp.wait()

        def group_body(g, _):
            def row_body(j, _):
                r = g * L + j
                acc = UP_v[r, pl.ds(0, L)] * IP_v[r, pl.ds(0, L)]
                for k in range(1, KU):
                    acc += UP_v[r, pl.ds(k * L, L)] * IP_v[r, pl.ds(k * L, L)]
                for f in range(N_FEAT):
                    for k in range(FEAT_DIM // L):
                        d = f * FEAT_DIM + k * L
                        acc += (FU_v[f * CHUNK + r, pl.ds(d, L)]
                                * FI_v[r, pl.ds(d, L)])
                sbuf_v[pl.ds(j * L, L)] = acc
                return 0

            lax.fori_loop(0, L, row_body, 0)

            t = mean_v[...]
            for d in range(L):
                t += plsc.load_gather(sbuf_v, [lanes * L + d])
            out_v[pl.ds(g * L, L)] = t
            return 0

        lax.fori_loop(0, GROUPS, group_body, 0)
        pltpu.sync_copy(out_v, out_h.at[pl.ds(cb, CHUNK)])
        return 0

    lax.fori_loop(0, NCHUNK, chunk_body, 0)


@jax.jit
def _sc_call(uid, iid, ufeat, up, ip, fup, fip, mean16):
    mesh = plsc.VectorSubcoreMesh(core_axis_name="c", subcore_axis_name="s",
                                  num_cores=NC, num_subcores=NS)
    f = pl.kernel(
        _sc_body,
        out_type=jax.ShapeDtypeStruct((B,), jnp.float32),
        mesh=mesh,
        compiler_params=pltpu.CompilerParams(needs_layout_passes=False,
                                             use_tc_tiling_on_sc=True),
        scratch_types=[
            pltpu.VMEM((CHUNK,), jnp.int32),
            pltpu.VMEM((CHUNK,), jnp.int32),
            pltpu.VMEM((CHUNK * N_FEAT,), jnp.int32),
            pltpu.VMEM((CHUNK * N_FEAT,), jnp.int32),
            pltpu.VMEM((CHUNK, WU), jnp.float32),
            pltpu.VMEM((CHUNK, WU), jnp.float32),
            pltpu.VMEM((CHUNK * N_FEAT, WF), jnp.float32),
            pltpu.VMEM((CHUNK, WF), jnp.float32),
            pltpu.VMEM((L,), jnp.float32),
            pltpu.VMEM((L * L,), jnp.float32),
            pltpu.VMEM((CHUNK,), jnp.float32),
            pltpu.SemaphoreType.DMA,
        ],
    )
    return f(uid, iid, ufeat, up, ip, fup, fip, mean16)


def kernel(uid, iid, u_feat, user_emb, user_bias, item_emb_mf, item_bias,
           feat_u, feat_i, mean, vae_mean, item_emb_lat):
    ones = jnp.ones((NUM_USERS, 1), jnp.float32)
    zpad = jnp.zeros((NUM_USERS, WU - 2 * EMB - 2), jnp.float32)
    up = jnp.concatenate([user_emb, vae_mean, user_bias, ones, zpad], axis=1)
    ip = jnp.concatenate([item_emb_mf, item_emb_lat, ones, item_bias, zpad],
                         axis=1)
    fup = jnp.concatenate([feat_u[0], feat_u[1], feat_u[2], feat_u[3]], axis=1)
    fip = jnp.concatenate([feat_i[0], feat_i[1], feat_i[2], feat_i[3]], axis=1)
    return _sc_call(uid, iid, u_feat.reshape(-1), up, ip, fup, fip,
                    jnp.broadcast_to(mean, (L,)))


# speedup vs baseline: 1.8400x; 1.8400x over previous
import functools

import jax
import jax.numpy as jnp
from jax import lax
from jax.experimental import pallas as pl
from jax.experimental.pallas import tpu as pltpu
from jax.experimental.pallas import tpu_sc as plsc

NUM_USERS = 100000
NUM_ITEMS = 100000
EMB = 64
N_FEAT = 4
FEAT_VOCAB = 1000
FEAT_DIM = 32
LATENT = 64
B = 16384

NC, NS, L = 2, 16, 16
NW = NC * NS
BPW = B // NW
CHUNK = 128
NCHUNK = BPW // CHUNK
GROUPS = CHUNK // L

_SET_TYPES = [
    pltpu.VMEM((CHUNK,), jnp.int32),
    pltpu.VMEM((CHUNK,), jnp.int32),
    pltpu.VMEM((N_FEAT, CHUNK), jnp.int32),
    pltpu.VMEM((CHUNK, EMB), jnp.float32),
    pltpu.VMEM((CHUNK, LATENT), jnp.float32),
    pltpu.VMEM((CHUNK, EMB), jnp.float32),
    pltpu.VMEM((CHUNK, LATENT), jnp.float32),
    pltpu.VMEM((N_FEAT, CHUNK, FEAT_DIM), jnp.float32),
    pltpu.VMEM((N_FEAT, CHUNK, FEAT_DIM), jnp.float32),
    pltpu.VMEM((CHUNK,), jnp.float32),
    pltpu.VMEM((CHUNK,), jnp.float32),
]
_NSET = len(_SET_TYPES)


def _sc_body(uid_h, iid_h, uft_h, ue_h, ub_h, mf_h, ib_h, fu_h, fi_h,
             mean_h, va_h, lat_h, part_h, bias_h, *scratch):
    setA = scratch[:_NSET]
    mean_v, part_v, biasout_v, sem0 = scratch[_NSET:]
    bufsets = (setA,)
    sems = (sem0,)

    wid = lax.axis_index("s") * NC + lax.axis_index("c")
    base = wid * BPW

    pltpu.sync_copy(mean_h, mean_v)

    def issue(ci, p):
        (uid_v, iid_v, fidx_v, UE_v, VA_v, MF_v, LAT_v,
         FU_v, FI_v, UB_v, IB_v) = bufsets[p]
        sem = sems[p]
        cb = base + ci * CHUNK
        pltpu.sync_copy(uid_h.at[pl.ds(cb, CHUNK)], uid_v)
        pltpu.sync_copy(iid_h.at[pl.ds(cb, CHUNK)], iid_v)
        for f in range(N_FEAT):
            pltpu.sync_copy(uft_h.at[pl.ds(f * B + cb, CHUNK)], fidx_v.at[f])

        cps = [
            pltpu.async_copy(ue_h.at[uid_v], UE_v, sem),
            pltpu.async_copy(va_h.at[uid_v], VA_v, sem),
            pltpu.async_copy(mf_h.at[iid_v], MF_v, sem),
            pltpu.async_copy(lat_h.at[iid_v], LAT_v, sem),
            pltpu.async_copy(ub_h.at[uid_v], UB_v, sem),
            pltpu.async_copy(ib_h.at[iid_v], IB_v, sem),
        ]
        for f in range(N_FEAT):
            cps.append(pltpu.async_copy(fu_h.at[f].at[fidx_v.at[f]],
                                        FU_v.at[f], sem))
            cps.append(pltpu.async_copy(fi_h.at[f].at[iid_v],
                                        FI_v.at[f], sem))
        return cps

    def compute(ci, p):
        (uid_v, iid_v, fidx_v, UE_v, VA_v, MF_v, LAT_v,
         FU_v, FI_v, UB_v, IB_v) = bufsets[p]
        cb = base + ci * CHUNK

        def row_body(r, _):
            acc = UE_v[r, pl.ds(0, L)] * MF_v[r, pl.ds(0, L)]
            for k in range(1, EMB // L):
                acc += UE_v[r, pl.ds(k * L, L)] * MF_v[r, pl.ds(k * L, L)]
            for k in range(LATENT // L):
                acc += VA_v[r, pl.ds(k * L, L)] * LAT_v[r, pl.ds(k * L, L)]
            for f in range(N_FEAT):
                for k in range(FEAT_DIM // L):
                    acc += (FU_v[f, r, pl.ds(k * L, L)]
                            * FI_v[f, r, pl.ds(k * L, L)])
            part_v[pl.ds(r * L, L)] = acc
            return 0

        lax.fori_loop(0, CHUNK, row_body, 0)

        def bias_body(g, _):
            biasout_v[pl.ds(g * L, L)] = (mean_v[...]
                                          + UB_v[pl.ds(g * L, L)]
                                          + IB_v[pl.ds(g * L, L)])
            return 0

        lax.fori_loop(0, GROUPS, bias_body, 0)

        pltpu.sync_copy(part_v, part_h.at[pl.ds(cb * L, CHUNK * L)])
        pltpu.sync_copy(biasout_v, bias_h.at[pl.ds(cb, CHUNK)])

    def chunk_body(ci, _):
        cps = issue(ci, 0)
        for cp in cps:
            cp.wait()
        compute(ci, 0)
        return 0

    lax.fori_loop(0, NCHUNK, chunk_body, 0)


TC_BLK = 2048


def _tc_body(p_ref, b_ref, o_ref):
    o_ref[...] = jnp.sum(p_ref[...], axis=1) + b_ref[...]


@jax.jit
def _call(uid, iid, ufeat_t, ue, ub, mf, ib, fu, fi, mean16, va, lat):
    mesh = plsc.VectorSubcoreMesh(core_axis_name="c", subcore_axis_name="s",
                                  num_cores=NC, num_subcores=NS)
    sc = pl.kernel(
        _sc_body,
        out_type=[jax.ShapeDtypeStruct((B * L,), jnp.float32),
                  jax.ShapeDtypeStruct((B,), jnp.float32)],
        mesh=mesh,
        compiler_params=pltpu.CompilerParams(use_tc_tiling_on_sc=False),
        scratch_types=_SET_TYPES + [
            pltpu.VMEM((L,), jnp.float32),
            pltpu.VMEM((CHUNK * L,), jnp.float32),
            pltpu.VMEM((CHUNK,), jnp.float32),
            pltpu.SemaphoreType.DMA,
        ],
    )
    partials, biases = sc(uid, iid, ufeat_t, ue, ub, mf, ib, fu, fi,
                          mean16, va, lat)
    out = pl.pallas_call(
        _tc_body,
        out_shape=jax.ShapeDtypeStruct((B,), jnp.float32),
        grid=(B // TC_BLK,),
        in_specs=[
            pl.BlockSpec((TC_BLK, L), lambda i: (i, 0)),
            pl.BlockSpec((TC_BLK,), lambda i: (i,)),
        ],
        out_specs=pl.BlockSpec((TC_BLK,), lambda i: (i,)),
    )(partials.reshape(B, L), biases)
    return out


def kernel(uid, iid, u_feat, user_emb, user_bias, item_emb_mf, item_bias,
           feat_u, feat_i, mean, vae_mean, item_emb_lat):
    return _call(uid, iid, u_feat.T.reshape(-1), user_emb,
                 user_bias.reshape(-1), item_emb_mf, item_bias.reshape(-1),
                 feat_u, feat_i,
                 jnp.broadcast_to(mean, (L,)), vae_mean, item_emb_lat)
